# baseline (speedup 1.0000x reference)
import math

import jax
import jax.numpy as jnp
from jax import lax
from jax.experimental import pallas as pl

N, DIM = 2048, 768
H, KQ, VAL, K = 12, 64, 64, 32
RN = 512
RQ = 512
NT_DIMS = (((1,), (1,)), ((), ()))


def _qkv_kernel(x_ref, wq_ref, bq_ref, wk_ref, bk_ref, wv_ref, bv_ref,
                q_ref, k_ref, v_ref):
    xb = x_ref[...]
    qf = jnp.dot(xb, wq_ref[...], preferred_element_type=jnp.float32) + bq_ref[...]
    kf = jnp.dot(xb, wk_ref[...], preferred_element_type=jnp.float32) + bk_ref[...]
    vf = jnp.dot(xb, wv_ref[...], preferred_element_type=jnp.float32) + bv_ref[...]
    for h in range(H):
        q_ref[h, :, :] = qf[:, h * KQ:(h + 1) * KQ]
        k_ref[h, :, :] = kf[:, h * KQ:(h + 1) * KQ]
        v_ref[h, :, :] = vf[:, h * VAL:(h + 1) * VAL]


def _attn_kernel(q_ref, k_ref, v_ref, o_ref):
    qb = q_ref[0]
    kb = k_ref[0]
    s = lax.dot_general(qb, kb, NT_DIMS,
                        preferred_element_type=jnp.float32)
    s = s * (1.0 / math.sqrt(KQ))

    u = lax.bitcast_convert_type(s, jnp.uint32)
    big = jnp.uint32(0x80000000)
    ukey = jnp.where(u >= big, ~u, u | big)

    rows = qb.shape[0]
    lo0 = jnp.zeros((rows, 1), jnp.uint32)
    hi0 = jnp.full((rows, 1), 0xFFFFFFFF, jnp.uint32)
    res0 = jnp.zeros((rows, 1), jnp.float32)
    ones_red = jnp.ones((s.shape[1], 8), jnp.float32)

    def body(_, carry):
        lo, hi = carry
        mid = lo + ((hi - lo) >> 1)
        maskf = (ukey >= mid).astype(jnp.float32)
        cnt = jnp.dot(maskf, ones_red,
                      preferred_element_type=jnp.float32)[:, :1]
        ok = cnt >= K
        return jnp.where(ok, mid, lo), jnp.where(ok, hi, mid)

    lo, hi = lax.fori_loop(0, 32, body, (lo0, hi0))

    fu = jnp.where(lo >= big, lo & jnp.uint32(0x7FFFFFFF), ~lo)
    tf = lax.bitcast_convert_type(fu, jnp.float32)
    p = jnp.where(ukey >= lo, jnp.exp(s - tf), 0.0)
    vv = jnp.concatenate(
        [v_ref[0], jnp.ones((kb.shape[0], 1), jnp.float32)], axis=1)
    op = jnp.dot(p, vv, preferred_element_type=jnp.float32)
    o_ref[0] = op[:, :VAL] / op[:, VAL:VAL + 1]


def _proj_kernel(a_ref, w_ref, b_ref, o_ref):
    acc = jnp.broadcast_to(b_ref[...], (a_ref.shape[1], DIM))
    for h in range(H):
        acc = acc + jnp.dot(a_ref[h], w_ref[h],
                            preferred_element_type=jnp.float32)
    o_ref[...] = acc


def kernel(x, WQ, bQ, WK, bK, WV, bV, WO, bO):
    x2 = x.reshape(N, DIM)
    full = lambda a, b: pl.BlockSpec((a, b), lambda *_: (0, 0))

    q, k, v = pl.pallas_call(
        _qkv_kernel,
        grid=(N // RN,),
        in_specs=[
            pl.BlockSpec((RN, DIM), lambda i: (i, 0)),
            full(DIM, H * KQ), full(1, H * KQ),
            full(DIM, H * KQ), full(1, H * KQ),
            full(DIM, H * VAL), full(1, H * VAL),
        ],
        out_specs=[
            pl.BlockSpec((H, RN, KQ), lambda i: (0, i, 0)),
            pl.BlockSpec((H, RN, KQ), lambda i: (0, i, 0)),
            pl.BlockSpec((H, RN, VAL), lambda i: (0, i, 0)),
        ],
        out_shape=[jax.ShapeDtypeStruct((H, N, KQ), jnp.float32)] * 2
        + [jax.ShapeDtypeStruct((H, N, VAL), jnp.float32)],
    )(x2, WQ, bQ[None, :], WK, bK[None, :], WV, bV[None, :])

    attn = pl.pallas_call(
        _attn_kernel,
        grid=(H, N // RQ),
        in_specs=[
            pl.BlockSpec((1, RQ, KQ), lambda h, i: (h, i, 0)),
            pl.BlockSpec((1, N, KQ), lambda h, i: (h, 0, 0)),
            pl.BlockSpec((1, N, VAL), lambda h, i: (h, 0, 0)),
        ],
        out_specs=pl.BlockSpec((1, RQ, VAL), lambda h, i: (h, i, 0)),
        out_shape=jax.ShapeDtypeStruct((H, N, VAL), jnp.float32),
    )(q, k, v)

    out = pl.pallas_call(
        _proj_kernel,
        grid=(N // RN,),
        in_specs=[
            pl.BlockSpec((H, RN, VAL), lambda i: (0, i, 0)),
            pl.BlockSpec((H, VAL, DIM), lambda i: (0, 0, 0)),
            full(1, DIM),
        ],
        out_specs=pl.BlockSpec((RN, DIM), lambda i: (i, 0)),
        out_shape=jax.ShapeDtypeStruct((N, DIM), jnp.float32),
    )(attn, WO.reshape(H, VAL, DIM), bO[None, :])
    return out.reshape(1, N, DIM)


# speedup vs baseline: 1.3304x; 1.3304x over previous
import math

import jax
import jax.numpy as jnp
from jax import lax
from jax.experimental import pallas as pl

N, DIM = 2048, 768
H, KQ, VAL, K = 12, 64, 64, 32
RN = 512
RQ = 512
NT_DIMS = (((1,), (1,)), ((), ()))


def _qkv_kernel(x_ref, wq_ref, bq_ref, wk_ref, bk_ref, wv_ref, bv_ref,
                q_ref, k_ref, v_ref):
    xb = x_ref[...]
    qf = jnp.dot(xb, wq_ref[...], preferred_element_type=jnp.float32) + bq_ref[...]
    kf = jnp.dot(xb, wk_ref[...], preferred_element_type=jnp.float32) + bk_ref[...]
    vf = jnp.dot(xb, wv_ref[...], preferred_element_type=jnp.float32) + bv_ref[...]
    for h in range(H):
        q_ref[h, :, :] = qf[:, h * KQ:(h + 1) * KQ]
        k_ref[h, :, :] = kf[:, h * KQ:(h + 1) * KQ]
        v_ref[h, :, :] = vf[:, h * VAL:(h + 1) * VAL]


def _attn_kernel(q_ref, k_ref, v_ref, o_ref):
    qb = q_ref[0]
    kb = k_ref[0]
    s = lax.dot_general(qb, kb, NT_DIMS,
                        preferred_element_type=jnp.float32)
    s = s * (1.0 / math.sqrt(KQ))

    u = lax.bitcast_convert_type(s, jnp.uint32)
    big = jnp.uint32(0x80000000)
    ukey = jnp.where(u >= big, ~u, u | big)

    rows = qb.shape[0]
    lo0 = jnp.zeros((rows, 1), jnp.uint32)
    hi0 = jnp.full((rows, 1), 0xFFFFFFFF, jnp.uint32)
    res0 = jnp.zeros((rows, 1), jnp.float32)
    ones_red = jnp.ones((s.shape[1], 8), jnp.float32)

    def cond(carry):
        it, _, _, resolved = carry
        return jnp.logical_and(it < 32, jnp.min(resolved) < 0.5)

    def body(carry):
        it, lo, hi, resolved = carry
        mid = lo + ((hi - lo) >> 1)
        cnt = jnp.sum((ukey >= mid).astype(jnp.int32), axis=1, keepdims=True)
        ok = cnt >= K
        lo = jnp.where(ok, mid, lo)
        hi = jnp.where(ok, hi, mid)
        resolved = jnp.where(ok, (cnt == K).astype(jnp.float32), resolved)
        return it + 1, lo, hi, resolved

    _, lo, hi, _ = lax.while_loop(cond, body, (0, lo0, hi0, res0))

    fu = jnp.where(lo >= big, lo & jnp.uint32(0x7FFFFFFF), ~lo)
    tf = lax.bitcast_convert_type(fu, jnp.float32)
    p = jnp.where(ukey >= lo, jnp.exp(s - tf), 0.0)
    vv = jnp.concatenate(
        [v_ref[0], jnp.ones((kb.shape[0], 1), jnp.float32)], axis=1)
    op = jnp.dot(p, vv, preferred_element_type=jnp.float32)
    o_ref[0] = op[:, :VAL] / op[:, VAL:VAL + 1]


def _proj_kernel(a_ref, w_ref, b_ref, o_ref):
    acc = jnp.broadcast_to(b_ref[...], (a_ref.shape[1], DIM))
    for h in range(H):
        acc = acc + jnp.dot(a_ref[h], w_ref[h],
                            preferred_element_type=jnp.float32)
    o_ref[...] = acc


def kernel(x, WQ, bQ, WK, bK, WV, bV, WO, bO):
    x2 = x.reshape(N, DIM)
    full = lambda a, b: pl.BlockSpec((a, b), lambda *_: (0, 0))

    q, k, v = pl.pallas_call(
        _qkv_kernel,
        grid=(N // RN,),
        in_specs=[
            pl.BlockSpec((RN, DIM), lambda i: (i, 0)),
            full(DIM, H * KQ), full(1, H * KQ),
            full(DIM, H * KQ), full(1, H * KQ),
            full(DIM, H * VAL), full(1, H * VAL),
        ],
        out_specs=[
            pl.BlockSpec((H, RN, KQ), lambda i: (0, i, 0)),
            pl.BlockSpec((H, RN, KQ), lambda i: (0, i, 0)),
            pl.BlockSpec((H, RN, VAL), lambda i: (0, i, 0)),
        ],
        out_shape=[jax.ShapeDtypeStruct((H, N, KQ), jnp.float32)] * 2
        + [jax.ShapeDtypeStruct((H, N, VAL), jnp.float32)],
    )(x2, WQ, bQ[None, :], WK, bK[None, :], WV, bV[None, :])

    attn = pl.pallas_call(
        _attn_kernel,
        grid=(H, N // RQ),
        in_specs=[
            pl.BlockSpec((1, RQ, KQ), lambda h, i: (h, i, 0)),
            pl.BlockSpec((1, N, KQ), lambda h, i: (h, 0, 0)),
            pl.BlockSpec((1, N, VAL), lambda h, i: (h, 0, 0)),
        ],
        out_specs=pl.BlockSpec((1, RQ, VAL), lambda h, i: (h, i, 0)),
        out_shape=jax.ShapeDtypeStruct((H, N, VAL), jnp.float32),
    )(q, k, v)

    out = pl.pallas_call(
        _proj_kernel,
        grid=(N // RN,),
        in_specs=[
            pl.BlockSpec((H, RN, VAL), lambda i: (0, i, 0)),
            pl.BlockSpec((H, VAL, DIM), lambda i: (0, 0, 0)),
            full(1, DIM),
        ],
        out_specs=pl.BlockSpec((RN, DIM), lambda i: (i, 0)),
        out_shape=jax.ShapeDtypeStruct((N, DIM), jnp.float32),
    )(attn, WO.reshape(H, VAL, DIM), bO[None, :])
    return out.reshape(1, N, DIM)


# speedup vs baseline: 1.7254x; 1.2969x over previous
import math

import jax
import jax.numpy as jnp
from jax import lax
from jax.experimental import pallas as pl

N, DIM = 2048, 768
H, KQ, VAL, K = 12, 64, 64, 32
RN = 512
RQ = 512
NT_DIMS = (((1,), (1,)), ((), ()))


def _qkv_kernel(x_ref, wq_ref, bq_ref, wk_ref, bk_ref, wv_ref, bv_ref,
                q_ref, k_ref, v_ref):
    xb = x_ref[...]
    qf = jnp.dot(xb, wq_ref[...], preferred_element_type=jnp.float32) + bq_ref[...]
    kf = jnp.dot(xb, wk_ref[...], preferred_element_type=jnp.float32) + bk_ref[...]
    vf = jnp.dot(xb, wv_ref[...], preferred_element_type=jnp.float32) + bv_ref[...]
    for h in range(H):
        q_ref[h, :, :] = qf[:, h * KQ:(h + 1) * KQ]
        k_ref[h, :, :] = kf[:, h * KQ:(h + 1) * KQ]
        v_ref[h, :, :] = vf[:, h * VAL:(h + 1) * VAL]


def _attn_kernel(q_ref, k_ref, v_ref, o_ref):
    qb = q_ref[0]
    kb = k_ref[0]
    s = lax.dot_general(qb, kb, NT_DIMS,
                        preferred_element_type=jnp.float32)
    s = s * (1.0 / math.sqrt(KQ))

    u = lax.bitcast_convert_type(s, jnp.uint32)
    big = jnp.uint32(0x80000000)
    ukey = jnp.where(u >= big, ~u, u | big)

    rows = qb.shape[0]
    m128 = s[:, :128]
    for c in range(1, 16):
        m128 = jnp.maximum(m128, s[:, c * 128:(c + 1) * 128])
    m64 = jnp.maximum(m128[:, :64], m128[:, 64:])
    t0 = jnp.min(m64, axis=1, keepdims=True)
    rmax = jnp.max(m64, axis=1, keepdims=True)

    def enc(f):
        uu = lax.bitcast_convert_type(f, jnp.uint32)
        return jnp.where(uu >= big, ~uu, uu | big)

    lo0 = enc(t0)
    hi0 = enc(rmax) + 1
    res0 = jnp.zeros((rows, 1), jnp.float32)

    def cond(carry):
        it, _, _, resolved = carry
        return jnp.logical_and(it < 32, jnp.min(resolved) < 0.5)

    def body(carry):
        it, lo, hi, resolved = carry
        mid = lo + ((hi - lo) >> 1)
        cnt = jnp.sum((ukey >= mid).astype(jnp.int32), axis=1, keepdims=True)
        ok = cnt >= K
        lo = jnp.where(ok, mid, lo)
        hi = jnp.where(ok, hi, mid)
        resolved = jnp.where(ok, (cnt == K).astype(jnp.float32), resolved)
        return it + 1, lo, hi, resolved

    _, lo, hi, _ = lax.while_loop(cond, body, (0, lo0, hi0, res0))

    fu = jnp.where(lo >= big, lo & jnp.uint32(0x7FFFFFFF), ~lo)
    tf = lax.bitcast_convert_type(fu, jnp.float32)
    p = jnp.where(ukey >= lo, jnp.exp(s - tf), 0.0)
    vv = jnp.concatenate(
        [v_ref[0], jnp.ones((kb.shape[0], 1), jnp.float32)], axis=1)
    op = jnp.dot(p, vv, preferred_element_type=jnp.float32)
    o_ref[0] = op[:, :VAL] / op[:, VAL:VAL + 1]


def _proj_kernel(a_ref, w_ref, b_ref, o_ref):
    acc = jnp.broadcast_to(b_ref[...], (a_ref.shape[1], DIM))
    for h in range(H):
        acc = acc + jnp.dot(a_ref[h], w_ref[h],
                            preferred_element_type=jnp.float32)
    o_ref[...] = acc


def kernel(x, WQ, bQ, WK, bK, WV, bV, WO, bO):
    x2 = x.reshape(N, DIM)
    full = lambda a, b: pl.BlockSpec((a, b), lambda *_: (0, 0))

    q, k, v = pl.pallas_call(
        _qkv_kernel,
        grid=(N // RN,),
        in_specs=[
            pl.BlockSpec((RN, DIM), lambda i: (i, 0)),
            full(DIM, H * KQ), full(1, H * KQ),
            full(DIM, H * KQ), full(1, H * KQ),
            full(DIM, H * VAL), full(1, H * VAL),
        ],
        out_specs=[
            pl.BlockSpec((H, RN, KQ), lambda i: (0, i, 0)),
            pl.BlockSpec((H, RN, KQ), lambda i: (0, i, 0)),
            pl.BlockSpec((H, RN, VAL), lambda i: (0, i, 0)),
        ],
        out_shape=[jax.ShapeDtypeStruct((H, N, KQ), jnp.float32)] * 2
        + [jax.ShapeDtypeStruct((H, N, VAL), jnp.float32)],
    )(x2, WQ, bQ[None, :], WK, bK[None, :], WV, bV[None, :])

    attn = pl.pallas_call(
        _attn_kernel,
        grid=(H, N // RQ),
        in_specs=[
            pl.BlockSpec((1, RQ, KQ), lambda h, i: (h, i, 0)),
            pl.BlockSpec((1, N, KQ), lambda h, i: (h, 0, 0)),
            pl.BlockSpec((1, N, VAL), lambda h, i: (h, 0, 0)),
        ],
        out_specs=pl.BlockSpec((1, RQ, VAL), lambda h, i: (h, i, 0)),
        out_shape=jax.ShapeDtypeStruct((H, N, VAL), jnp.float32),
    )(q, k, v)

    out = pl.pallas_call(
        _proj_kernel,
        grid=(N // RN,),
        in_specs=[
            pl.BlockSpec((H, RN, VAL), lambda i: (0, i, 0)),
            pl.BlockSpec((H, VAL, DIM), lambda i: (0, 0, 0)),
            full(1, DIM),
        ],
        out_specs=pl.BlockSpec((RN, DIM), lambda i: (i, 0)),
        out_shape=jax.ShapeDtypeStruct((N, DIM), jnp.float32),
    )(attn, WO.reshape(H, VAL, DIM), bO[None, :])
    return out.reshape(1, N, DIM)


# speedup vs baseline: 1.7678x; 1.0246x over previous
import math

import jax
import jax.numpy as jnp
from jax import lax
from jax.experimental import pallas as pl

N, DIM = 2048, 768
H, KQ, VAL, K = 12, 64, 64, 32
RN = 512
RQ = 512
NT_DIMS = (((1,), (1,)), ((), ()))


def _qkv_kernel(x_ref, wq_ref, bq_ref, wk_ref, bk_ref, wv_ref, bv_ref,
                q_ref, k_ref, v_ref):
    xb = x_ref[...]
    qf = jnp.dot(xb, wq_ref[...], preferred_element_type=jnp.float32) + bq_ref[...]
    kf = jnp.dot(xb, wk_ref[...], preferred_element_type=jnp.float32) + bk_ref[...]
    vf = jnp.dot(xb, wv_ref[...], preferred_element_type=jnp.float32) + bv_ref[...]
    for h in range(H):
        q_ref[h, :, :] = qf[:, h * KQ:(h + 1) * KQ]
        k_ref[h, :, :] = kf[:, h * KQ:(h + 1) * KQ]
        v_ref[h, :, :] = vf[:, h * VAL:(h + 1) * VAL]


def _attn_kernel(q_ref, k_ref, v_ref, o_ref):
    qb = q_ref[0]
    kb = k_ref[0]
    s = lax.dot_general(qb, kb, NT_DIMS,
                        preferred_element_type=jnp.float32)
    s = s * (1.0 / math.sqrt(KQ))

    u = lax.bitcast_convert_type(s, jnp.uint32)
    big = jnp.uint32(0x80000000)
    ukey = jnp.where(u >= big, ~u, u | big)

    rows = qb.shape[0]
    m128 = s[:, :128]
    for c in range(1, 16):
        m128 = jnp.maximum(m128, s[:, c * 128:(c + 1) * 128])
    m64 = jnp.maximum(m128[:, :64], m128[:, 64:])
    m32 = jnp.maximum(m64[:, :32], m64[:, 32:])
    t0 = jnp.min(m32, axis=1, keepdims=True)
    rmax = jnp.max(m32, axis=1, keepdims=True)

    def enc(f):
        uu = lax.bitcast_convert_type(f, jnp.uint32)
        return jnp.where(uu >= big, ~uu, uu | big)

    lo0 = enc(t0)
    hi0 = enc(rmax) + 1
    res0 = jnp.zeros((rows, 1), jnp.float32)

    def cond(carry):
        it, _, _, resolved = carry
        return jnp.logical_and(it < 32, jnp.min(resolved) < 0.5)

    def body(carry):
        it, lo, hi, resolved = carry
        mid = lo + ((hi - lo) >> 1)
        cnt = jnp.sum((ukey >= mid).astype(jnp.int32), axis=1, keepdims=True)
        ok = cnt >= K
        lo = jnp.where(ok, mid, lo)
        hi = jnp.where(ok, hi, mid)
        resolved = jnp.where(ok, (cnt == K).astype(jnp.float32), resolved)
        return it + 1, lo, hi, resolved

    _, lo, hi, _ = lax.while_loop(cond, body, (0, lo0, hi0, res0))

    fu = jnp.where(lo >= big, lo & jnp.uint32(0x7FFFFFFF), ~lo)
    tf = lax.bitcast_convert_type(fu, jnp.float32)
    p = jnp.where(ukey >= lo, jnp.exp(s - tf), 0.0)
    vv = jnp.concatenate(
        [v_ref[0], jnp.ones((kb.shape[0], 1), jnp.float32)], axis=1)
    op = jnp.dot(p.astype(jnp.bfloat16), vv.astype(jnp.bfloat16),
                 preferred_element_type=jnp.float32)
    o_ref[0] = op[:, :VAL] / op[:, VAL:VAL + 1]


def _proj_kernel(a_ref, w_ref, b_ref, o_ref):
    acc = jnp.broadcast_to(b_ref[...], (a_ref.shape[1], DIM))
    for h in range(H):
        acc = acc + jnp.dot(a_ref[h], w_ref[h],
                            preferred_element_type=jnp.float32)
    o_ref[...] = acc


def kernel(x, WQ, bQ, WK, bK, WV, bV, WO, bO):
    x2 = x.reshape(N, DIM)
    full = lambda a, b: pl.BlockSpec((a, b), lambda *_: (0, 0))

    q, k, v = pl.pallas_call(
        _qkv_kernel,
        grid=(N // RN,),
        in_specs=[
            pl.BlockSpec((RN, DIM), lambda i: (i, 0)),
            full(DIM, H * KQ), full(1, H * KQ),
            full(DIM, H * KQ), full(1, H * KQ),
            full(DIM, H * VAL), full(1, H * VAL),
        ],
        out_specs=[
            pl.BlockSpec((H, RN, KQ), lambda i: (0, i, 0)),
            pl.BlockSpec((H, RN, KQ), lambda i: (0, i, 0)),
            pl.BlockSpec((H, RN, VAL), lambda i: (0, i, 0)),
        ],
        out_shape=[jax.ShapeDtypeStruct((H, N, KQ), jnp.float32)] * 2
        + [jax.ShapeDtypeStruct((H, N, VAL), jnp.float32)],
    )(x2, WQ, bQ[None, :], WK, bK[None, :], WV, bV[None, :])

    attn = pl.pallas_call(
        _attn_kernel,
        grid=(H, N // RQ),
        in_specs=[
            pl.BlockSpec((1, RQ, KQ), lambda h, i: (h, i, 0)),
            pl.BlockSpec((1, N, KQ), lambda h, i: (h, 0, 0)),
            pl.BlockSpec((1, N, VAL), lambda h, i: (h, 0, 0)),
        ],
        out_specs=pl.BlockSpec((1, RQ, VAL), lambda h, i: (h, i, 0)),
        out_shape=jax.ShapeDtypeStruct((H, N, VAL), jnp.float32),
    )(q, k, v)

    out = pl.pallas_call(
        _proj_kernel,
        grid=(N // RN,),
        in_specs=[
            pl.BlockSpec((H, RN, VAL), lambda i: (0, i, 0)),
            pl.BlockSpec((H, VAL, DIM), lambda i: (0, 0, 0)),
            full(1, DIM),
        ],
        out_specs=pl.BlockSpec((RN, DIM), lambda i: (i, 0)),
        out_shape=jax.ShapeDtypeStruct((N, DIM), jnp.float32),
    )(attn, WO.reshape(H, VAL, DIM), bO[None, :])
    return out.reshape(1, N, DIM)
